# Initial kernel scaffold; baseline (speedup 1.0000x reference)
import jax
import jax.numpy as jnp
from jax.experimental import pallas as pl


def kernel(user_emb, item_emb, pad_aspect, in_proj_w, in_proj_b, out_proj_w, out_proj_b, edge_index):
    raise NotImplementedError("write your pallas kernel here")


# speedup vs baseline: 18.5109x; 18.5109x over previous
import functools
import math

import jax
import jax.numpy as jnp
import numpy as np
from jax import lax
from jax.experimental import pallas as pl
from jax.experimental.pallas import tpu as pltpu

try:
    from jax.experimental.pallas import tpu_sc as plsc
except ImportError:
    plsc = None

N_USERS = 30000
N_ITEMS = 20000
N = N_USERS + N_ITEMS
E = 800000
D = 64
HEADS = 4
DH = D // HEADS
LAYERS = 3
MAXLEN = 5

NP = 51200
BN = 2048
NBLK = NP // BN

_INTERPRET = False


def _layer_body(scale_acc,
                agg_ref, din_ref, dout_ref,
                a0_ref, a1_ref, a2_ref, a3_ref, a4_ref,
                wqt_ref, wkt_ref, wvt_ref, bq_ref, bk_ref, bv_ref,
                wot_ref, bo_ref, hmat_ref, emat_ref, acc_ref,
                accout_ref, hnext_ref):
    f32 = jnp.float32
    agg = agg_ref[...]
    norm_in = lax.rsqrt(jnp.maximum(din_ref[...], f32(1.0)))
    rst = agg * norm_in
    q = jnp.dot(rst, wqt_ref[...], preferred_element_type=f32) + bq_ref[...]
    hmat = hmat_ref[...]
    emat = emat_ref[...]
    wkt = wkt_ref[...]
    wvt = wvt_ref[...]
    bk = bk_ref[...]
    bv = bv_ref[...]
    asp_refs = (a0_ref, a1_ref, a2_ref, a3_ref, a4_ref)
    inv = f32(1.0 / math.sqrt(DH))
    s_list = []
    v_list = []
    for k in range(MAXLEN):
        a_k = asp_refs[k][...]
        kk = jnp.dot(a_k, wkt, preferred_element_type=f32) + bk
        vk = jnp.dot(a_k, wvt, preferred_element_type=f32) + bv
        sk = jnp.dot(q * kk, hmat, preferred_element_type=f32) * inv
        s_list.append(sk)
        v_list.append(vk)
    m = s_list[0]
    for k in range(1, MAXLEN):
        m = jnp.maximum(m, s_list[k])
    znum = jnp.zeros_like(agg)
    zden = jnp.zeros_like(m)
    for k in range(MAXLEN):
        ek = jnp.exp(s_list[k] - m)
        zden = zden + ek
        znum = znum + jnp.dot(ek, emat, preferred_element_type=f32) * v_list[k]
    o = znum / jnp.dot(zden, emat, preferred_element_type=f32)
    asp_out = jnp.dot(o, wot_ref[...], preferred_element_type=f32) + bo_ref[...]
    emb_new = (rst + asp_out) * f32(0.5)
    accout_ref[...] = (acc_ref[...] + emb_new) * f32(scale_acc)
    norm_out = lax.rsqrt(jnp.maximum(dout_ref[...], f32(1.0)))
    hnext_ref[...] = emb_new * norm_out


def _tc_layer(agg, deg_in, deg_out, asp_list, weights, acc, last):
    (wqt, wkt, wvt, bq, bk, bv, wot, bo, hmat, emat) = weights
    _z = np.int32(0)
    row = lambda i: (i, _z)
    zero = lambda i: (_z, _z)
    specs = [
        pl.BlockSpec((BN, D), row),
        pl.BlockSpec((BN, 1), row),
        pl.BlockSpec((BN, 1), row),
    ]
    specs += [pl.BlockSpec((BN, D), row)] * MAXLEN
    specs += [
        pl.BlockSpec((D, D), zero),
        pl.BlockSpec((D, D), zero),
        pl.BlockSpec((D, D), zero),
        pl.BlockSpec((1, D), zero),
        pl.BlockSpec((1, D), zero),
        pl.BlockSpec((1, D), zero),
        pl.BlockSpec((D, D), zero),
        pl.BlockSpec((1, D), zero),
        pl.BlockSpec((D, HEADS), zero),
        pl.BlockSpec((HEADS, D), zero),
        pl.BlockSpec((BN, D), row),
    ]
    out_specs = [pl.BlockSpec((BN, D), row), pl.BlockSpec((BN, D), row)]
    out_shapes = [jax.ShapeDtypeStruct((NP, D), jnp.float32),
                  jax.ShapeDtypeStruct((NP, D), jnp.float32)]
    scale = 0.25 if last else 1.0
    fn = pl.pallas_call(
        functools.partial(_layer_body, scale),
        grid=(NBLK,),
        in_specs=specs,
        out_specs=out_specs,
        out_shape=out_shapes,
        interpret=_INTERPRET,
    )
    return fn(agg, deg_in, deg_out, *asp_list, wqt, wkt, wvt, bq, bk, bv,
              wot, bo, hmat, emat, acc)


def _prep_body(emb_ref, dout_ref, h_ref):
    norm_out = lax.rsqrt(jnp.maximum(dout_ref[...], jnp.float32(1.0)))
    h_ref[...] = emb_ref[...] * norm_out


def _tc_prep(emb0, deg_out):
    row = lambda i: (i, np.int32(0))
    return pl.pallas_call(
        _prep_body,
        grid=(NBLK,),
        in_specs=[pl.BlockSpec((BN, D), row), pl.BlockSpec((BN, 1), row)],
        out_specs=pl.BlockSpec((BN, D), row),
        out_shape=jax.ShapeDtypeStruct((NP, D), jnp.float32),
        interpret=_INTERPRET,
    )(emb0, deg_out)


def _bincounts(src, dst):
    deg_out = jnp.zeros((NP,), jnp.float32).at[src].add(1.0)
    deg_in = jnp.zeros((NP,), jnp.float32).at[dst].add(1.0)
    return deg_out, deg_in


def _segsum(h, src, dst):
    return jnp.zeros((NP, D), jnp.float32).at[dst].add(h[src])


def kernel(user_emb, item_emb, pad_aspect, in_proj_w, in_proj_b,
           out_proj_w, out_proj_b, edge_index):
    f32 = jnp.float32
    src = edge_index[0].astype(jnp.int32)
    dst = edge_index[1].astype(jnp.int32)
    emb0 = jnp.concatenate([user_emb.astype(f32), item_emb.astype(f32)], axis=0)
    emb0 = jnp.pad(emb0, ((0, NP - N), (0, 0)))
    asp_list = [pad_aspect[:, k, :].astype(f32) for k in range(MAXLEN)]

    wq = in_proj_w[:D].astype(f32)
    wk = in_proj_w[D:2 * D].astype(f32)
    wv = in_proj_w[2 * D:].astype(f32)
    bq = in_proj_b[:D].astype(f32).reshape(1, D)
    bk = in_proj_b[D:2 * D].astype(f32).reshape(1, D)
    bv = in_proj_b[2 * D:].astype(f32).reshape(1, D)
    hmat_np = np.zeros((D, HEADS), np.float32)
    for h_ in range(HEADS):
        hmat_np[h_ * DH:(h_ + 1) * DH, h_] = 1.0
    hmat = jnp.asarray(hmat_np)
    emat = jnp.asarray(np.ascontiguousarray(hmat_np.T))
    weights = (wq.T, wk.T, wv.T, bq, bk, bv,
               out_proj_w.astype(f32).T, out_proj_b.astype(f32).reshape(1, D),
               hmat, emat)

    deg_out, deg_in = _bincounts(src, dst)
    deg_out2 = deg_out.reshape(NP, 1)
    deg_in2 = deg_in.reshape(NP, 1)

    h = _tc_prep(emb0, deg_out2)
    acc = emb0
    for layer in range(LAYERS):
        agg = _segsum(h, src, dst)
        acc, h = _tc_layer(agg, deg_in2, deg_out2, asp_list, weights, acc,
                           last=(layer == LAYERS - 1))
    light = acc[:N]
    return (light[:N_USERS], light[N_USERS:])


# speedup vs baseline: 71.6665x; 3.8716x over previous
import functools
import math

import jax
import jax.numpy as jnp
import numpy as np
from jax import lax
from jax.experimental import pallas as pl
from jax.experimental.pallas import tpu as pltpu

try:
    from jax.experimental.pallas import tpu_sc as plsc
except ImportError:
    plsc = None

N_USERS = 30000
N_ITEMS = 20000
N = N_USERS + N_ITEMS
E = 800000
D = 64
HEADS = 4
DH = D // HEADS
LAYERS = 3
MAXLEN = 5

NP = 51200
BN = 2048
NBLK = NP // BN

_INTERPRET = False


def _layer_body(scale_acc,
                agg_ref, din_ref, dout_ref,
                a0_ref, a1_ref, a2_ref, a3_ref, a4_ref,
                wqt_ref, wkt_ref, wvt_ref, bq_ref, bk_ref, bv_ref,
                wot_ref, bo_ref, hmat_ref, emat_ref, acc_ref,
                accout_ref, hnext_ref):
    f32 = jnp.float32
    agg = agg_ref[...]
    norm_in = lax.rsqrt(jnp.maximum(din_ref[...], f32(1.0)))
    rst = agg * norm_in
    q = jnp.dot(rst, wqt_ref[...], preferred_element_type=f32) + bq_ref[...]
    hmat = hmat_ref[...]
    emat = emat_ref[...]
    wkt = wkt_ref[...]
    wvt = wvt_ref[...]
    bk = bk_ref[...]
    bv = bv_ref[...]
    asp_refs = (a0_ref, a1_ref, a2_ref, a3_ref, a4_ref)
    inv = f32(1.0 / math.sqrt(DH))
    s_list = []
    v_list = []
    for k in range(MAXLEN):
        a_k = asp_refs[k][...]
        kk = jnp.dot(a_k, wkt, preferred_element_type=f32) + bk
        vk = jnp.dot(a_k, wvt, preferred_element_type=f32) + bv
        sk = jnp.dot(q * kk, hmat, preferred_element_type=f32) * inv
        s_list.append(sk)
        v_list.append(vk)
    m = s_list[0]
    for k in range(1, MAXLEN):
        m = jnp.maximum(m, s_list[k])
    znum = jnp.zeros_like(agg)
    zden = jnp.zeros_like(m)
    for k in range(MAXLEN):
        ek = jnp.exp(s_list[k] - m)
        zden = zden + ek
        znum = znum + jnp.dot(ek, emat, preferred_element_type=f32) * v_list[k]
    o = znum / jnp.dot(zden, emat, preferred_element_type=f32)
    asp_out = jnp.dot(o, wot_ref[...], preferred_element_type=f32) + bo_ref[...]
    emb_new = (rst + asp_out) * f32(0.5)
    accout_ref[...] = (acc_ref[...] + emb_new) * f32(scale_acc)
    norm_out = lax.rsqrt(jnp.maximum(dout_ref[...], f32(1.0)))
    hnext_ref[...] = emb_new * norm_out


def _tc_layer(agg, deg_in, deg_out, asp_list, weights, acc, last):
    (wqt, wkt, wvt, bq, bk, bv, wot, bo, hmat, emat) = weights
    _z = np.int32(0)
    row = lambda i: (i, _z)
    zero = lambda i: (_z, _z)
    specs = [
        pl.BlockSpec((BN, D), row),
        pl.BlockSpec((BN, 1), row),
        pl.BlockSpec((BN, 1), row),
    ]
    specs += [pl.BlockSpec((BN, D), row)] * MAXLEN
    specs += [
        pl.BlockSpec((D, D), zero),
        pl.BlockSpec((D, D), zero),
        pl.BlockSpec((D, D), zero),
        pl.BlockSpec((1, D), zero),
        pl.BlockSpec((1, D), zero),
        pl.BlockSpec((1, D), zero),
        pl.BlockSpec((D, D), zero),
        pl.BlockSpec((1, D), zero),
        pl.BlockSpec((D, HEADS), zero),
        pl.BlockSpec((HEADS, D), zero),
        pl.BlockSpec((BN, D), row),
    ]
    out_specs = [pl.BlockSpec((BN, D), row), pl.BlockSpec((BN, D), row)]
    out_shapes = [jax.ShapeDtypeStruct((NP, D), jnp.float32),
                  jax.ShapeDtypeStruct((NP, D), jnp.float32)]
    scale = 0.25 if last else 1.0
    fn = pl.pallas_call(
        functools.partial(_layer_body, scale),
        grid=(NBLK,),
        in_specs=specs,
        out_specs=out_specs,
        out_shape=out_shapes,
        interpret=_INTERPRET,
    )
    return fn(agg, deg_in, deg_out, *asp_list, wqt, wkt, wvt, bq, bk, bv,
              wot, bo, hmat, emat, acc)


def _prep_body(emb_ref, dout_ref, h_ref):
    norm_out = lax.rsqrt(jnp.maximum(dout_ref[...], jnp.float32(1.0)))
    h_ref[...] = emb_ref[...] * norm_out


def _tc_prep(emb0, deg_out):
    row = lambda i: (i, np.int32(0))
    return pl.pallas_call(
        _prep_body,
        grid=(NBLK,),
        in_specs=[pl.BlockSpec((BN, D), row), pl.BlockSpec((BN, 1), row)],
        out_specs=pl.BlockSpec((BN, D), row),
        out_shape=jax.ShapeDtypeStruct((NP, D), jnp.float32),
        interpret=_INTERPRET,
    )(emb0, deg_out)


_TILE_E = E // 16
_STG = 2000
_GB = 80
_NB = _STG // _GB
_HALF = NP // 2
_TROWS = _HALF // 16
_DUMP = 32
_ZCH = 64

_CNT_T = NP // 16
_CNT_CH = 640


def _zero_vec16():
    return jnp.zeros((16,), jnp.float32)


def _fill_rows_zero(buf, rows):
    for r in range(rows):
        for k in range(D // 16):
            buf[r, pl.ds(k * 16, 16)] = _zero_vec16()


def _segsum_body(h_hbm, src_hbm, dst_hbm, out_hbm,
                 srcstg, dststg, idxbuf, rows_a, rows_b, zbuf, sem_g, sem_s,
                 acc):
    c = lax.axis_index("c")
    s = lax.axis_index("s")
    base = c * jnp.int32(_HALF)
    tile_e0 = s * jnp.int32(_TILE_E)

    _fill_rows_zero(zbuf, _ZCH)
    for j in range(_TROWS // _ZCH):
        pltpu.sync_copy(zbuf, acc.at[pl.ds(s * jnp.int32(_TROWS) + jnp.int32(j * _ZCH), _ZCH)])
    plsc.subcore_barrier()

    def compute_idx(off):
        for j in range(_GB // 16):
            dv = dststg[pl.ds(off + jnp.int32(j * 16), 16)]
            local = dv - base
            owned = (dv >= base) & (local < jnp.int32(_HALF))
            dump = jnp.int32(_HALF) + (dv & jnp.int32(_DUMP - 1))
            idxbuf[pl.ds(j * 16, 16)] = jnp.where(owned, local, dump)

    def batch(gi, buf):
        off = gi * jnp.int32(_GB)
        cp = pltpu.async_copy(h_hbm.at[srcstg.at[pl.ds(off, _GB)]], buf, sem_g)
        compute_idx(off)
        cp.wait()
        pltpu.sync_copy(buf, acc.at[idxbuf], add=True)

    for ri in range(_TILE_E // _STG):
        stg0 = tile_e0 + jnp.int32(ri * _STG)
        pltpu.sync_copy(src_hbm.at[pl.ds(stg0, _STG)], srcstg)
        pltpu.sync_copy(dst_hbm.at[pl.ds(stg0, _STG)], dststg)

        def body2(gi, _):
            g2 = gi * jnp.int32(2)
            batch(g2, rows_a)
            batch(g2 + jnp.int32(1), rows_b)
            return jnp.int32(0)

        lax.fori_loop(jnp.int32(0), jnp.int32(_NB // 2), body2, jnp.int32(0))
        if _NB % 2:
            batch(jnp.int32(_NB - 1), rows_a)

    plsc.subcore_barrier()

    for j in range(_TROWS // _ZCH):
        r0 = s * jnp.int32(_TROWS) + jnp.int32(j * _ZCH)
        pltpu.sync_copy(acc.at[pl.ds(r0, _ZCH)], zbuf)
        pltpu.sync_copy(zbuf, out_hbm.at[pl.ds(base + r0, _ZCH)])


def _segsum(h, src, dst):
    mesh = plsc.VectorSubcoreMesh(core_axis_name="c", subcore_axis_name="s")
    fn = pl.kernel(
        _segsum_body,
        out_type=jax.ShapeDtypeStruct((NP, D), jnp.float32),
        mesh=mesh,
        scratch_types=[
            pltpu.VMEM((_STG,), jnp.int32),
            pltpu.VMEM((_STG,), jnp.int32),
            pltpu.VMEM((_GB,), jnp.int32),
            pltpu.VMEM((_GB, D), jnp.float32),
            pltpu.VMEM((_GB, D), jnp.float32),
            pltpu.VMEM((_ZCH, D), jnp.float32),
            pltpu.SemaphoreType.DMA,
            pltpu.SemaphoreType.DMA,
            pltpu.VMEM_SHARED((_HALF + _DUMP, D), jnp.float32),
        ],
        compiler_params=pltpu.CompilerParams(use_tc_tiling_on_sc=False),
    )
    return fn(h, src, dst)


def _bincount_half(idx_hbm, out_hbm, stg, idxbuf, ones_v, zbuf, dbuf, cnt, s):
    def zb(_i, _):
        zbuf[pl.ds(_i * jnp.int32(16), 16)] = _zero_vec16()
        return jnp.int32(0)
    lax.fori_loop(jnp.int32(0), jnp.int32(_CNT_CH // 16), zb, jnp.int32(0))
    for j in range(_CNT_T // _CNT_CH):
        pltpu.sync_copy(zbuf, cnt.at[pl.ds(s * jnp.int32(_CNT_T) + jnp.int32(j * _CNT_CH), _CNT_CH)])
    plsc.subcore_barrier()

    for k in range(_GB // 16):
        ones_v[pl.ds(k * 16, 16)] = jnp.ones((16,), jnp.float32)

    tile_e0 = s * jnp.int32(_TILE_E)
    for ri in range(_TILE_E // _STG):
        pltpu.sync_copy(idx_hbm.at[pl.ds(tile_e0 + jnp.int32(ri * _STG), _STG)], stg)

        def body(gi, _):
            off = gi * jnp.int32(_GB)
            for j in range(_GB // 16):
                idxbuf[pl.ds(j * 16, 16)] = stg[pl.ds(off + jnp.int32(j * 16), 16)]
            pltpu.sync_copy(ones_v, cnt.at[idxbuf], add=True)
            return jnp.int32(0)

        lax.fori_loop(jnp.int32(0), jnp.int32(_NB), body, jnp.int32(0))

    plsc.subcore_barrier()
    for j in range(_CNT_T // _CNT_CH):
        o = s * jnp.int32(_CNT_T) + jnp.int32(j * _CNT_CH)
        pltpu.sync_copy(cnt.at[pl.ds(o, _CNT_CH)], dbuf)
        pltpu.sync_copy(dbuf, out_hbm.at[pl.ds(o, _CNT_CH)])


def _bincount_body(src_hbm, dst_hbm, dego_hbm, degi_hbm,
                   stg, idxbuf, ones_v, zbuf, dbuf, sem, cnt):
    c = lax.axis_index("c")
    s = lax.axis_index("s")

    @pl.when(c == 0)
    def _():
        _bincount_half(src_hbm, dego_hbm, stg, idxbuf, ones_v, zbuf, dbuf,
                       cnt, s)

    @pl.when(c == 1)
    def _():
        _bincount_half(dst_hbm, degi_hbm, stg, idxbuf, ones_v, zbuf, dbuf,
                       cnt, s)


def _bincounts(src, dst):
    mesh = plsc.VectorSubcoreMesh(core_axis_name="c", subcore_axis_name="s")
    fn = pl.kernel(
        _bincount_body,
        out_type=[jax.ShapeDtypeStruct((NP,), jnp.float32),
                  jax.ShapeDtypeStruct((NP,), jnp.float32)],
        mesh=mesh,
        scratch_types=[
            pltpu.VMEM((_STG,), jnp.int32),
            pltpu.VMEM((_GB,), jnp.int32),
            pltpu.VMEM((_GB,), jnp.float32),
            pltpu.VMEM((_CNT_CH,), jnp.float32),
            pltpu.VMEM((_CNT_CH,), jnp.float32),
            pltpu.SemaphoreType.DMA,
            pltpu.VMEM_SHARED((NP,), jnp.float32),
        ],
        compiler_params=pltpu.CompilerParams(use_tc_tiling_on_sc=False),
    )
    return fn(src, dst)


def kernel(user_emb, item_emb, pad_aspect, in_proj_w, in_proj_b,
           out_proj_w, out_proj_b, edge_index):
    f32 = jnp.float32
    src = edge_index[0].astype(jnp.int32)
    dst = edge_index[1].astype(jnp.int32)
    emb0 = jnp.concatenate([user_emb.astype(f32), item_emb.astype(f32)], axis=0)
    emb0 = jnp.pad(emb0, ((0, NP - N), (0, 0)))
    asp_list = [pad_aspect[:, k, :].astype(f32) for k in range(MAXLEN)]

    wq = in_proj_w[:D].astype(f32)
    wk = in_proj_w[D:2 * D].astype(f32)
    wv = in_proj_w[2 * D:].astype(f32)
    bq = in_proj_b[:D].astype(f32).reshape(1, D)
    bk = in_proj_b[D:2 * D].astype(f32).reshape(1, D)
    bv = in_proj_b[2 * D:].astype(f32).reshape(1, D)
    hmat_np = np.zeros((D, HEADS), np.float32)
    for h_ in range(HEADS):
        hmat_np[h_ * DH:(h_ + 1) * DH, h_] = 1.0
    hmat = jnp.asarray(hmat_np)
    emat = jnp.asarray(np.ascontiguousarray(hmat_np.T))
    weights = (wq.T, wk.T, wv.T, bq, bk, bv,
               out_proj_w.astype(f32).T, out_proj_b.astype(f32).reshape(1, D),
               hmat, emat)

    deg_out, deg_in = _bincounts(src, dst)
    deg_out2 = deg_out.reshape(NP, 1)
    deg_in2 = deg_in.reshape(NP, 1)

    h = _tc_prep(emb0, deg_out2)
    acc = emb0
    for layer in range(LAYERS):
        agg = _segsum(h, src, dst)
        acc, h = _tc_layer(agg, deg_in2, deg_out2, asp_list, weights, acc,
                           last=(layer == LAYERS - 1))
    light = acc[:N]
    return (light[:N_USERS], light[N_USERS:])


# speedup vs baseline: 116.0394x; 1.6192x over previous
import functools
import math

import jax
import jax.numpy as jnp
import numpy as np
from jax import lax
from jax.experimental import pallas as pl
from jax.experimental.pallas import tpu as pltpu

try:
    from jax.experimental.pallas import tpu_sc as plsc
except ImportError:
    plsc = None

N_USERS = 30000
N_ITEMS = 20000
N = N_USERS + N_ITEMS
E = 800000
D = 64
HEADS = 4
DH = D // HEADS
LAYERS = 3
MAXLEN = 5

NP = 51200
BN = 2048
NBLK = NP // BN

_INTERPRET = False


def _layer_body(scale_acc,
                agg_ref, din_ref, dout_ref,
                a0_ref, a1_ref, a2_ref, a3_ref, a4_ref,
                wqt_ref, wkt_ref, wvt_ref, bq_ref, bk_ref, bv_ref,
                wot_ref, bo_ref, hmat_ref, emat_ref, acc_ref,
                accout_ref, hnext_ref):
    f32 = jnp.float32
    agg = agg_ref[...]
    norm_in = lax.rsqrt(jnp.maximum(din_ref[...], f32(1.0)))
    rst = agg * norm_in
    q = jnp.dot(rst, wqt_ref[...], preferred_element_type=f32) + bq_ref[...]
    hmat = hmat_ref[...]
    emat = emat_ref[...]
    wkt = wkt_ref[...]
    wvt = wvt_ref[...]
    bk = bk_ref[...]
    bv = bv_ref[...]
    asp_refs = (a0_ref, a1_ref, a2_ref, a3_ref, a4_ref)
    inv = f32(1.0 / math.sqrt(DH))
    s_list = []
    v_list = []
    for k in range(MAXLEN):
        a_k = asp_refs[k][...]
        kk = jnp.dot(a_k, wkt, preferred_element_type=f32) + bk
        vk = jnp.dot(a_k, wvt, preferred_element_type=f32) + bv
        sk = jnp.dot(q * kk, hmat, preferred_element_type=f32) * inv
        s_list.append(sk)
        v_list.append(vk)
    m = s_list[0]
    for k in range(1, MAXLEN):
        m = jnp.maximum(m, s_list[k])
    znum = jnp.zeros_like(agg)
    zden = jnp.zeros_like(m)
    for k in range(MAXLEN):
        ek = jnp.exp(s_list[k] - m)
        zden = zden + ek
        znum = znum + jnp.dot(ek, emat, preferred_element_type=f32) * v_list[k]
    o = znum / jnp.dot(zden, emat, preferred_element_type=f32)
    asp_out = jnp.dot(o, wot_ref[...], preferred_element_type=f32) + bo_ref[...]
    emb_new = (rst + asp_out) * f32(0.5)
    accout_ref[...] = (acc_ref[...] + emb_new) * f32(scale_acc)
    norm_out = lax.rsqrt(jnp.maximum(dout_ref[...], f32(1.0)))
    hnext_ref[...] = emb_new * norm_out


def _tc_layer(agg, deg_in, deg_out, asp_list, weights, acc, last):
    (wqt, wkt, wvt, bq, bk, bv, wot, bo, hmat, emat) = weights
    _z = np.int32(0)
    row = lambda i: (i, _z)
    zero = lambda i: (_z, _z)
    specs = [
        pl.BlockSpec((BN, D), row),
        pl.BlockSpec((BN, 1), row),
        pl.BlockSpec((BN, 1), row),
    ]
    specs += [pl.BlockSpec((BN, D), row)] * MAXLEN
    specs += [
        pl.BlockSpec((D, D), zero),
        pl.BlockSpec((D, D), zero),
        pl.BlockSpec((D, D), zero),
        pl.BlockSpec((1, D), zero),
        pl.BlockSpec((1, D), zero),
        pl.BlockSpec((1, D), zero),
        pl.BlockSpec((D, D), zero),
        pl.BlockSpec((1, D), zero),
        pl.BlockSpec((D, HEADS), zero),
        pl.BlockSpec((HEADS, D), zero),
        pl.BlockSpec((BN, D), row),
    ]
    out_specs = [pl.BlockSpec((BN, D), row), pl.BlockSpec((BN, D), row)]
    out_shapes = [jax.ShapeDtypeStruct((NP, D), jnp.float32),
                  jax.ShapeDtypeStruct((NP, D), jnp.float32)]
    scale = 0.25 if last else 1.0
    fn = pl.pallas_call(
        functools.partial(_layer_body, scale),
        grid=(NBLK,),
        in_specs=specs,
        out_specs=out_specs,
        out_shape=out_shapes,
        interpret=_INTERPRET,
    )
    return fn(agg, deg_in, deg_out, *asp_list, wqt, wkt, wvt, bq, bk, bv,
              wot, bo, hmat, emat, acc)


def _prep_body(emb_ref, dout_ref, h_ref):
    norm_out = lax.rsqrt(jnp.maximum(dout_ref[...], jnp.float32(1.0)))
    h_ref[...] = emb_ref[...] * norm_out


def _tc_prep(emb0, deg_out):
    row = lambda i: (i, np.int32(0))
    return pl.pallas_call(
        _prep_body,
        grid=(NBLK,),
        in_specs=[pl.BlockSpec((BN, D), row), pl.BlockSpec((BN, 1), row)],
        out_specs=pl.BlockSpec((BN, D), row),
        out_shape=jax.ShapeDtypeStruct((NP, D), jnp.float32),
        interpret=_INTERPRET,
    )(emb0, deg_out)


_TILE_E = E // 16
_STG = 2000
_GB = 80
_NB = _STG // _GB
_EP = 819200
_ETILE = _EP // 16
_ESTG = 2560
_EGB = 128
_ENB = _ESTG // _EGB
_HALF = NP // 2
_TROWS = _HALF // 16
_DUMP = 32
_ZCH = 64

_CNT_T = NP // 16
_CNT_CH = 640


def _zero_vec16():
    return jnp.zeros((16,), jnp.float32)


def _fill_rows_zero(buf, rows):
    for r in range(rows):
        for k in range(D // 16):
            buf[r, pl.ds(k * 16, 16)] = _zero_vec16()


def _segsum_body(h_hbm, src_hbm, dst_hbm, out_hbm,
                 srcstg, dststg, idx2, rows_a, rows_b, zbuf,
                 sem_ga, sem_gb, sem_sa, sem_sb,
                 acc):
    c = lax.axis_index("c")
    s = lax.axis_index("s")
    base = c * jnp.int32(_HALF)
    tile_e0 = s * jnp.int32(_ETILE)

    _fill_rows_zero(zbuf, _ZCH)
    for j in range(_TROWS // _ZCH):
        pltpu.sync_copy(zbuf, acc.at[pl.ds(s * jnp.int32(_TROWS) + jnp.int32(j * _ZCH), _ZCH)])
    plsc.subcore_barrier()

    rows = (rows_a, rows_b)
    gsem = (sem_ga, sem_gb)
    ssem = (sem_sa, sem_sb)

    def compute_idx(b, slot):
        for j in range(_EGB // 16):
            dv = dststg[pl.ds(b * _EGB + j * 16, 16)]
            local = dv - base
            owned = (dv >= base) & (local < jnp.int32(_HALF))
            dump = jnp.int32(_HALF) + (dv & jnp.int32(_DUMP - 1))
            idx2[slot, pl.ds(j * 16, 16)] = jnp.where(owned, local, dump)

    def start_gather(b, slot):
        return pltpu.async_copy(
            h_hbm.at[srcstg.at[pl.ds(b * _EGB, _EGB)]], rows[slot], gsem[slot])

    def start_scatter(slot):
        return pltpu.async_copy(
            rows[slot], acc.at[idx2.at[np.int32(slot)]], ssem[slot], add=True)

    def refill(r, _):
        stg0 = tile_e0 + r * jnp.int32(_ESTG)
        pltpu.sync_copy(src_hbm.at[pl.ds(stg0, _ESTG)], srcstg)
        pltpu.sync_copy(dst_hbm.at[pl.ds(stg0, _ESTG)], dststg)
        cps = [None, None]
        scs = [None, None]
        cps[0] = start_gather(0, 0)
        for b in range(1, _ENB):
            sl = b % 2
            if scs[sl] is not None:
                scs[sl].wait()
            cps[sl] = start_gather(b, sl)
            p, ps = b - 1, (b - 1) % 2
            cps[ps].wait()
            compute_idx(p, ps)
            scs[ps] = start_scatter(ps)
        ps = (_ENB - 1) % 2
        cps[ps].wait()
        compute_idx(_ENB - 1, ps)
        scs[ps] = start_scatter(ps)
        scs[0].wait()
        scs[1].wait()
        return jnp.int32(0)

    lax.fori_loop(jnp.int32(0), jnp.int32(_ETILE // _ESTG), refill,
                  jnp.int32(0))

    plsc.subcore_barrier()

    for j in range(_TROWS // _ZCH):
        r0 = s * jnp.int32(_TROWS) + jnp.int32(j * _ZCH)
        pltpu.sync_copy(acc.at[pl.ds(r0, _ZCH)], zbuf)
        pltpu.sync_copy(zbuf, out_hbm.at[pl.ds(base + r0, _ZCH)])


def _segsum(h, src_p, dst_p):
    mesh = plsc.VectorSubcoreMesh(core_axis_name="c", subcore_axis_name="s")
    fn = pl.kernel(
        _segsum_body,
        out_type=jax.ShapeDtypeStruct((NP, D), jnp.float32),
        mesh=mesh,
        scratch_types=[
            pltpu.VMEM((_ESTG,), jnp.int32),
            pltpu.VMEM((_ESTG,), jnp.int32),
            pltpu.VMEM((2, _EGB), jnp.int32),
            pltpu.VMEM((_EGB, D), jnp.float32),
            pltpu.VMEM((_EGB, D), jnp.float32),
            pltpu.VMEM((_ZCH, D), jnp.float32),
            pltpu.SemaphoreType.DMA,
            pltpu.SemaphoreType.DMA,
            pltpu.SemaphoreType.DMA,
            pltpu.SemaphoreType.DMA,
            pltpu.VMEM_SHARED((_HALF + _DUMP, D), jnp.float32),
        ],
        compiler_params=pltpu.CompilerParams(use_tc_tiling_on_sc=False),
    )
    return fn(h, src_p, dst_p)


def _bincount_half(idx_hbm, out_hbm, stg, idxbuf, ones_v, zbuf, dbuf, cnt, s):
    def zb(_i, _):
        zbuf[pl.ds(_i * jnp.int32(16), 16)] = _zero_vec16()
        return jnp.int32(0)
    lax.fori_loop(jnp.int32(0), jnp.int32(_CNT_CH // 16), zb, jnp.int32(0))
    for j in range(_CNT_T // _CNT_CH):
        pltpu.sync_copy(zbuf, cnt.at[pl.ds(s * jnp.int32(_CNT_T) + jnp.int32(j * _CNT_CH), _CNT_CH)])
    plsc.subcore_barrier()

    for k in range(_GB // 16):
        ones_v[pl.ds(k * 16, 16)] = jnp.ones((16,), jnp.float32)

    tile_e0 = s * jnp.int32(_TILE_E)
    for ri in range(_TILE_E // _STG):
        pltpu.sync_copy(idx_hbm.at[pl.ds(tile_e0 + jnp.int32(ri * _STG), _STG)], stg)

        def body(gi, _):
            off = gi * jnp.int32(_GB)
            for j in range(_GB // 16):
                idxbuf[pl.ds(j * 16, 16)] = stg[pl.ds(off + jnp.int32(j * 16), 16)]
            pltpu.sync_copy(ones_v, cnt.at[idxbuf], add=True)
            return jnp.int32(0)

        lax.fori_loop(jnp.int32(0), jnp.int32(_NB), body, jnp.int32(0))

    plsc.subcore_barrier()
    for j in range(_CNT_T // _CNT_CH):
        o = s * jnp.int32(_CNT_T) + jnp.int32(j * _CNT_CH)
        pltpu.sync_copy(cnt.at[pl.ds(o, _CNT_CH)], dbuf)
        pltpu.sync_copy(dbuf, out_hbm.at[pl.ds(o, _CNT_CH)])


def _bincount_body(src_hbm, dst_hbm, dego_hbm, degi_hbm,
                   stg, idxbuf, ones_v, zbuf, dbuf, sem, cnt):
    c = lax.axis_index("c")
    s = lax.axis_index("s")

    @pl.when(c == 0)
    def _():
        _bincount_half(src_hbm, dego_hbm, stg, idxbuf, ones_v, zbuf, dbuf,
                       cnt, s)

    @pl.when(c == 1)
    def _():
        _bincount_half(dst_hbm, degi_hbm, stg, idxbuf, ones_v, zbuf, dbuf,
                       cnt, s)


def _bincounts(src, dst):
    mesh = plsc.VectorSubcoreMesh(core_axis_name="c", subcore_axis_name="s")
    fn = pl.kernel(
        _bincount_body,
        out_type=[jax.ShapeDtypeStruct((NP,), jnp.float32),
                  jax.ShapeDtypeStruct((NP,), jnp.float32)],
        mesh=mesh,
        scratch_types=[
            pltpu.VMEM((_STG,), jnp.int32),
            pltpu.VMEM((_GB,), jnp.int32),
            pltpu.VMEM((_GB,), jnp.float32),
            pltpu.VMEM((_CNT_CH,), jnp.float32),
            pltpu.VMEM((_CNT_CH,), jnp.float32),
            pltpu.SemaphoreType.DMA,
            pltpu.VMEM_SHARED((NP,), jnp.float32),
        ],
        compiler_params=pltpu.CompilerParams(use_tc_tiling_on_sc=False),
    )
    return fn(src, dst)


def kernel(user_emb, item_emb, pad_aspect, in_proj_w, in_proj_b,
           out_proj_w, out_proj_b, edge_index):
    f32 = jnp.float32
    src = edge_index[0].astype(jnp.int32)
    dst = edge_index[1].astype(jnp.int32)
    emb0 = jnp.concatenate([user_emb.astype(f32), item_emb.astype(f32)], axis=0)
    emb0 = jnp.pad(emb0, ((0, NP - N), (0, 0)))
    asp_list = [pad_aspect[:, k, :].astype(f32) for k in range(MAXLEN)]

    wq = in_proj_w[:D].astype(f32)
    wk = in_proj_w[D:2 * D].astype(f32)
    wv = in_proj_w[2 * D:].astype(f32)
    bq = in_proj_b[:D].astype(f32).reshape(1, D)
    bk = in_proj_b[D:2 * D].astype(f32).reshape(1, D)
    bv = in_proj_b[2 * D:].astype(f32).reshape(1, D)
    hmat_np = np.zeros((D, HEADS), np.float32)
    for h_ in range(HEADS):
        hmat_np[h_ * DH:(h_ + 1) * DH, h_] = 1.0
    hmat = jnp.asarray(hmat_np)
    emat = jnp.asarray(np.ascontiguousarray(hmat_np.T))
    weights = (wq.T, wk.T, wv.T, bq, bk, bv,
               out_proj_w.astype(f32).T, out_proj_b.astype(f32).reshape(1, D),
               hmat, emat)

    pad_n = _EP - E
    src_p = jnp.concatenate(
        [src, (jnp.arange(pad_n, dtype=jnp.int32) % jnp.int32(N))])
    dst_p = jnp.concatenate(
        [dst, jnp.full((pad_n,), -1, dtype=jnp.int32)])

    deg_out, deg_in = _bincounts(src, dst)
    deg_out2 = deg_out.reshape(NP, 1)
    deg_in2 = deg_in.reshape(NP, 1)

    h = _tc_prep(emb0, deg_out2)
    acc = emb0
    for layer in range(LAYERS):
        agg = _segsum(h, src_p, dst_p)
        acc, h = _tc_layer(agg, deg_in2, deg_out2, asp_list, weights, acc,
                           last=(layer == LAYERS - 1))
    light = acc[:N]
    return (light[:N_USERS], light[N_USERS:])


# speedup vs baseline: 135.1366x; 1.1646x over previous
import functools
import math

import jax
import jax.numpy as jnp
import numpy as np
from jax import lax
from jax.experimental import pallas as pl
from jax.experimental.pallas import tpu as pltpu

try:
    from jax.experimental.pallas import tpu_sc as plsc
except ImportError:
    plsc = None

N_USERS = 30000
N_ITEMS = 20000
N = N_USERS + N_ITEMS
E = 800000
D = 64
HEADS = 4
DH = D // HEADS
LAYERS = 3
MAXLEN = 5

NP = 51200
BN = 2048
NBLK = NP // BN

_INTERPRET = False


def _layer_body(scale_acc,
                agg_ref, din_ref, dout_ref,
                a0_ref, a1_ref, a2_ref, a3_ref, a4_ref,
                wqt_ref, wkt_ref, wvt_ref, bq_ref, bk_ref, bv_ref,
                wot_ref, bo_ref, hmat_ref, emat_ref, acc_ref,
                accout_ref, hnext_ref):
    f32 = jnp.float32
    agg = agg_ref[...]
    norm_in = lax.rsqrt(jnp.maximum(din_ref[...], f32(1.0)))
    rst = agg * norm_in
    q = jnp.dot(rst, wqt_ref[...], preferred_element_type=f32) + bq_ref[...]
    hmat = hmat_ref[...]
    emat = emat_ref[...]
    wkt = wkt_ref[...]
    wvt = wvt_ref[...]
    bk = bk_ref[...]
    bv = bv_ref[...]
    asp_refs = (a0_ref, a1_ref, a2_ref, a3_ref, a4_ref)
    inv = f32(1.0 / math.sqrt(DH))
    s_list = []
    v_list = []
    for k in range(MAXLEN):
        a_k = asp_refs[k][...]
        kk = jnp.dot(a_k, wkt, preferred_element_type=f32) + bk
        vk = jnp.dot(a_k, wvt, preferred_element_type=f32) + bv
        sk = jnp.dot(q * kk, hmat, preferred_element_type=f32) * inv
        s_list.append(sk)
        v_list.append(vk)
    m = s_list[0]
    for k in range(1, MAXLEN):
        m = jnp.maximum(m, s_list[k])
    znum = jnp.zeros_like(agg)
    zden = jnp.zeros_like(m)
    for k in range(MAXLEN):
        ek = jnp.exp(s_list[k] - m)
        zden = zden + ek
        znum = znum + jnp.dot(ek, emat, preferred_element_type=f32) * v_list[k]
    o = znum / jnp.dot(zden, emat, preferred_element_type=f32)
    asp_out = jnp.dot(o, wot_ref[...], preferred_element_type=f32) + bo_ref[...]
    emb_new = (rst + asp_out) * f32(0.5)
    accout_ref[...] = (acc_ref[...] + emb_new) * f32(scale_acc)
    norm_out = lax.rsqrt(jnp.maximum(dout_ref[...], f32(1.0)))
    hnext_ref[...] = emb_new * norm_out


def _tc_layer(agg, deg_in, deg_out, asp_list, weights, acc, last):
    (wqt, wkt, wvt, bq, bk, bv, wot, bo, hmat, emat) = weights
    _z = np.int32(0)
    row = lambda i: (i, _z)
    zero = lambda i: (_z, _z)
    specs = [
        pl.BlockSpec((BN, D), row),
        pl.BlockSpec((BN, 1), row),
        pl.BlockSpec((BN, 1), row),
    ]
    specs += [pl.BlockSpec((BN, D), row)] * MAXLEN
    specs += [
        pl.BlockSpec((D, D), zero),
        pl.BlockSpec((D, D), zero),
        pl.BlockSpec((D, D), zero),
        pl.BlockSpec((1, D), zero),
        pl.BlockSpec((1, D), zero),
        pl.BlockSpec((1, D), zero),
        pl.BlockSpec((D, D), zero),
        pl.BlockSpec((1, D), zero),
        pl.BlockSpec((D, HEADS), zero),
        pl.BlockSpec((HEADS, D), zero),
        pl.BlockSpec((BN, D), row),
    ]
    out_specs = [pl.BlockSpec((BN, D), row), pl.BlockSpec((BN, D), row)]
    out_shapes = [jax.ShapeDtypeStruct((NP, D), jnp.float32),
                  jax.ShapeDtypeStruct((NP, D), jnp.float32)]
    scale = 0.25 if last else 1.0
    fn = pl.pallas_call(
        functools.partial(_layer_body, scale),
        grid=(NBLK,),
        in_specs=specs,
        out_specs=out_specs,
        out_shape=out_shapes,
        interpret=_INTERPRET,
    )
    return fn(agg, deg_in, deg_out, *asp_list, wqt, wkt, wvt, bq, bk, bv,
              wot, bo, hmat, emat, acc)


def _prep_body(emb_ref, dout_ref, h_ref):
    norm_out = lax.rsqrt(jnp.maximum(dout_ref[...], jnp.float32(1.0)))
    h_ref[...] = emb_ref[...] * norm_out


def _tc_prep(emb0, deg_out):
    row = lambda i: (i, np.int32(0))
    return pl.pallas_call(
        _prep_body,
        grid=(NBLK,),
        in_specs=[pl.BlockSpec((BN, D), row), pl.BlockSpec((BN, 1), row)],
        out_specs=pl.BlockSpec((BN, D), row),
        out_shape=jax.ShapeDtypeStruct((NP, D), jnp.float32),
        interpret=_INTERPRET,
    )(emb0, deg_out)


_TILE_E = E // 16
_STG = 2000
_GB = 80
_NB = _STG // _GB
_EP = 819200
_ETILE = _EP // 16
_ESTG = 2560
_EGB = 128
_ENB = _ESTG // _EGB
_RTILE = _EP // 32
_RSTG = 2560
_RST = 25728
_RCORE = 32 * _RST
_RSZ = 2 * _RCORE + _ESTG
_FIFO = _ESTG + 128
_HALF = NP // 2
_TROWS = _HALF // 16
_DUMP = 32
_ZCH = 64

_CNT_T = NP // 16
_CNT_CH = 640


def _zero_vec16():
    return jnp.zeros((16,), jnp.float32)


def _fill_rows_zero(buf, rows):
    for r in range(rows):
        for k in range(D // 16):
            buf[r, pl.ds(k * 16, 16)] = _zero_vec16()


def _route_body(src_hbm, dst_hbm, rsrc_hbm, rdst_hbm, cnt_hbm,
                sstg, dstg, fs0, fd0, fs1, fd1, cbuf):
    c = lax.axis_index("c")
    s = lax.axis_index("s")
    t = c * jnp.int32(16) + s
    e0 = t * jnp.int32(_RTILE)
    rb0 = t * jnp.int32(_RST)
    rb1 = jnp.int32(_RCORE) + rb0
    ii = lax.iota(jnp.int32, 16)
    z = jnp.int32(0)

    def flush(F, off, fs, fd, rb):
        for j in range(_EGB // 16):
            sl = F + jnp.int32(j * 16) + ii
            plsc.store_scatter(fs, [sl], jnp.int32(j * 16) + ii)
            plsc.store_scatter(
                fd, [sl],
                jnp.int32(_HALF) + ((jnp.int32(j * 16) + ii)
                                    & jnp.int32(_DUMP - 1)))
        nblk = (jnp.max(F) + jnp.int32(_EGB - 1)) // jnp.int32(_EGB)

        def dr(b, _):
            bo = pl.multiple_of(b * jnp.int32(_EGB), 128)
            dsto = pl.multiple_of(rb + off + bo, 128)
            pltpu.sync_copy(fs.at[pl.ds(bo, _EGB)],
                            rsrc_hbm.at[pl.ds(dsto, _EGB)])
            pltpu.sync_copy(fd.at[pl.ds(bo, _EGB)],
                            rdst_hbm.at[pl.ds(dsto, _EGB)])
            return z

        lax.fori_loop(z, nblk, dr, z)
        return off + nblk * jnp.int32(_EGB)

    def refill(r, state):
        o0, o1 = state
        stg0 = pl.multiple_of(e0 + r * jnp.int32(_RSTG), 128)
        pltpu.sync_copy(src_hbm.at[pl.ds(stg0, _RSTG)], sstg)
        pltpu.sync_copy(dst_hbm.at[pl.ds(stg0, _RSTG)], dstg)
        f0 = jnp.zeros((16,), jnp.int32)
        f1 = jnp.zeros((16,), jnp.int32)
        for ch in range(_RSTG // 16):
            sv = sstg[pl.ds(ch * 16, 16)]
            dv = dstg[pl.ds(ch * 16, 16)]
            own0 = (dv >= jnp.int32(0)) & (dv < jnp.int32(_HALF))
            own1 = dv >= jnp.int32(_HALF)
            m0 = own0.astype(jnp.int32)
            m1 = own1.astype(jnp.int32)
            p0 = plsc.cumsum(m0)
            p1 = plsc.cumsum(m1)
            plsc.store_scatter(fs0, [f0 + p0 - m0], sv, mask=own0)
            plsc.store_scatter(fd0, [f0 + p0 - m0], dv, mask=own0)
            plsc.store_scatter(fs1, [f1 + p1 - m1], sv, mask=own1)
            plsc.store_scatter(fd1, [f1 + p1 - m1], dv - jnp.int32(_HALF),
                               mask=own1)
            f0 = f0 + plsc.all_reduce_population_count(own0)
            f1 = f1 + plsc.all_reduce_population_count(own1)
        o0 = flush(f0, o0, fs0, fd0, rb0)
        o1 = flush(f1, o1, fs1, fd1, rb1)
        return (o0, o1)

    o0, o1 = lax.fori_loop(z, jnp.int32(_RTILE // _RSTG), refill, (z, z))

    def wcnt(total, cslot):
        cbuf[pl.ds(0, 16)] = jnp.zeros((16,), jnp.int32) + total
        pltpu.sync_copy(
            cbuf, cnt_hbm.at[pl.ds(pl.multiple_of(cslot * 16, 16), 16)])

    wcnt(o0, t)
    wcnt(o1, jnp.int32(32) + t)


def _route(src_p, dst_p):
    mesh = plsc.VectorSubcoreMesh(core_axis_name="c", subcore_axis_name="s")
    fn = pl.kernel(
        _route_body,
        out_type=[jax.ShapeDtypeStruct((_RSZ,), jnp.int32),
                  jax.ShapeDtypeStruct((_RSZ,), jnp.int32),
                  jax.ShapeDtypeStruct((64 * 16,), jnp.int32)],
        mesh=mesh,
        scratch_types=[
            pltpu.VMEM((_RSTG,), jnp.int32),
            pltpu.VMEM((_RSTG,), jnp.int32),
            pltpu.VMEM((_FIFO,), jnp.int32),
            pltpu.VMEM((_FIFO,), jnp.int32),
            pltpu.VMEM((_FIFO,), jnp.int32),
            pltpu.VMEM((_FIFO,), jnp.int32),
            pltpu.VMEM((16,), jnp.int32),
        ],
        compiler_params=pltpu.CompilerParams(use_tc_tiling_on_sc=False,
                                             needs_layout_passes=False),
    )
    return fn(src_p, dst_p)


def _segsum_body(h_hbm, rsrc_hbm, rdst_hbm, cnt_hbm, out_hbm,
                 srcstg, dststg, idx2, rows_a, rows_b, zbuf, cbuf,
                 sem_ga, sem_gb, sem_sa, sem_sb,
                 acc):
    c = lax.axis_index("c")
    s = lax.axis_index("s")
    base = c * jnp.int32(_HALF)

    _fill_rows_zero(zbuf, _ZCH)
    for j in range(_TROWS // _ZCH):
        pltpu.sync_copy(zbuf, acc.at[pl.ds(s * jnp.int32(_TROWS) + jnp.int32(j * _ZCH), _ZCH)])
    plsc.subcore_barrier()

    rows = (rows_a, rows_b)
    gsem = (sem_ga, sem_gb)
    ssem = (sem_sa, sem_sb)
    z = jnp.int32(0)

    def fill_idx(off, slot):
        for j in range(_EGB // 16):
            idx2[slot, pl.ds(j * 16, 16)] = dststg[pl.ds(off + jnp.int32(j * 16), 16)]

    def start_gather(off, slot):
        return pltpu.async_copy(
            h_hbm.at[srcstg.at[pl.ds(pl.multiple_of(off, 128), _EGB)]], rows[slot], gsem[slot])

    def start_scatter(slot):
        return pltpu.async_copy(
            rows[slot], acc.at[idx2.at[np.int32(slot)]], ssem[slot], add=True)

    def do_batch(off, slot):
        g = start_gather(off, slot)
        g.wait()
        fill_idx(off, slot)
        sc = start_scatter(slot)
        return sc

    def region(tt, cslot):
        rb = c * jnp.int32(_RCORE) + tt * jnp.int32(_RST)
        pltpu.sync_copy(cnt_hbm.at[pl.ds(pl.multiple_of(cslot * 16, 16), 16)], cbuf)
        m = jnp.max(cbuf[pl.ds(0, 16)])
        nb = m // jnp.int32(_EGB)

        def refill(r, _):
            ro = pl.multiple_of(rb + r * jnp.int32(_ESTG), 128)
            pltpu.sync_copy(rsrc_hbm.at[pl.ds(ro, _ESTG)], srcstg)
            pltpu.sync_copy(rdst_hbm.at[pl.ds(ro, _ESTG)], dststg)
            nb_r = jnp.minimum(jnp.int32(_ENB), nb - r * jnp.int32(_ENB))

            def pair(i, _):
                o0 = i * jnp.int32(2 * _EGB)
                g0 = start_gather(o0, 0)
                g1 = start_gather(o0 + jnp.int32(_EGB), 1)
                g0.wait()
                fill_idx(o0, 0)
                s0 = start_scatter(0)
                g1.wait()
                fill_idx(o0 + jnp.int32(_EGB), 1)
                s1 = start_scatter(1)
                s0.wait()
                s1.wait()
                return z

            lax.fori_loop(z, nb_r // jnp.int32(2), pair, z)

            @pl.when(nb_r % jnp.int32(2) == jnp.int32(1))
            def _():
                sc = do_batch((nb_r - jnp.int32(1)) * jnp.int32(_EGB), 0)
                sc.wait()

            return z

        lax.fori_loop(z, (nb + jnp.int32(_ENB - 1)) // jnp.int32(_ENB),
                      refill, z)

    region(s * jnp.int32(2), c * jnp.int32(32) + s * jnp.int32(2))
    region(s * jnp.int32(2) + jnp.int32(1),
           c * jnp.int32(32) + s * jnp.int32(2) + jnp.int32(1))

    plsc.subcore_barrier()

    for j in range(_TROWS // _ZCH):
        r0 = s * jnp.int32(_TROWS) + jnp.int32(j * _ZCH)
        pltpu.sync_copy(acc.at[pl.ds(r0, _ZCH)], zbuf)
        pltpu.sync_copy(zbuf, out_hbm.at[pl.ds(base + r0, _ZCH)])


def _segsum(h, rsrc, rdst, cnt):
    mesh = plsc.VectorSubcoreMesh(core_axis_name="c", subcore_axis_name="s")
    fn = pl.kernel(
        _segsum_body,
        out_type=jax.ShapeDtypeStruct((NP, D), jnp.float32),
        mesh=mesh,
        scratch_types=[
            pltpu.VMEM((_ESTG,), jnp.int32),
            pltpu.VMEM((_ESTG,), jnp.int32),
            pltpu.VMEM((2, _EGB), jnp.int32),
            pltpu.VMEM((_EGB, D), jnp.float32),
            pltpu.VMEM((_EGB, D), jnp.float32),
            pltpu.VMEM((_ZCH, D), jnp.float32),
            pltpu.VMEM((16,), jnp.int32),
            pltpu.SemaphoreType.DMA,
            pltpu.SemaphoreType.DMA,
            pltpu.SemaphoreType.DMA,
            pltpu.SemaphoreType.DMA,
            pltpu.VMEM_SHARED((_HALF + _DUMP, D), jnp.float32),
        ],
        compiler_params=pltpu.CompilerParams(use_tc_tiling_on_sc=False,
                                             needs_layout_passes=False),
    )
    return fn(h, rsrc, rdst, cnt)


def _bincount_half(idx_hbm, out_hbm, stg, idxbuf, ones_v, zbuf, dbuf, cnt, s):
    def zb(_i, _):
        zbuf[pl.ds(_i * jnp.int32(16), 16)] = _zero_vec16()
        return jnp.int32(0)
    lax.fori_loop(jnp.int32(0), jnp.int32(_CNT_CH // 16), zb, jnp.int32(0))
    for j in range(_CNT_T // _CNT_CH):
        pltpu.sync_copy(zbuf, cnt.at[pl.ds(s * jnp.int32(_CNT_T) + jnp.int32(j * _CNT_CH), _CNT_CH)])
    plsc.subcore_barrier()

    for k in range(_GB // 16):
        ones_v[pl.ds(k * 16, 16)] = jnp.ones((16,), jnp.float32)

    tile_e0 = s * jnp.int32(_TILE_E)
    for ri in range(_TILE_E // _STG):
        pltpu.sync_copy(idx_hbm.at[pl.ds(tile_e0 + jnp.int32(ri * _STG), _STG)], stg)

        def body(gi, _):
            off = gi * jnp.int32(_GB)
            for j in range(_GB // 16):
                idxbuf[pl.ds(j * 16, 16)] = stg[pl.ds(off + jnp.int32(j * 16), 16)]
            pltpu.sync_copy(ones_v, cnt.at[idxbuf], add=True)
            return jnp.int32(0)

        lax.fori_loop(jnp.int32(0), jnp.int32(_NB), body, jnp.int32(0))

    plsc.subcore_barrier()
    for j in range(_CNT_T // _CNT_CH):
        o = s * jnp.int32(_CNT_T) + jnp.int32(j * _CNT_CH)
        pltpu.sync_copy(cnt.at[pl.ds(o, _CNT_CH)], dbuf)
        pltpu.sync_copy(dbuf, out_hbm.at[pl.ds(o, _CNT_CH)])


def _bincount_body(src_hbm, dst_hbm, dego_hbm, degi_hbm,
                   stg, idxbuf, ones_v, zbuf, dbuf, sem, cnt):
    c = lax.axis_index("c")
    s = lax.axis_index("s")

    @pl.when(c == 0)
    def _():
        _bincount_half(src_hbm, dego_hbm, stg, idxbuf, ones_v, zbuf, dbuf,
                       cnt, s)

    @pl.when(c == 1)
    def _():
        _bincount_half(dst_hbm, degi_hbm, stg, idxbuf, ones_v, zbuf, dbuf,
                       cnt, s)


def _bincounts(src, dst):
    mesh = plsc.VectorSubcoreMesh(core_axis_name="c", subcore_axis_name="s")
    fn = pl.kernel(
        _bincount_body,
        out_type=[jax.ShapeDtypeStruct((NP,), jnp.float32),
                  jax.ShapeDtypeStruct((NP,), jnp.float32)],
        mesh=mesh,
        scratch_types=[
            pltpu.VMEM((_STG,), jnp.int32),
            pltpu.VMEM((_GB,), jnp.int32),
            pltpu.VMEM((_GB,), jnp.float32),
            pltpu.VMEM((_CNT_CH,), jnp.float32),
            pltpu.VMEM((_CNT_CH,), jnp.float32),
            pltpu.SemaphoreType.DMA,
            pltpu.VMEM_SHARED((NP,), jnp.float32),
        ],
        compiler_params=pltpu.CompilerParams(use_tc_tiling_on_sc=False),
    )
    return fn(src, dst)


def kernel(user_emb, item_emb, pad_aspect, in_proj_w, in_proj_b,
           out_proj_w, out_proj_b, edge_index):
    f32 = jnp.float32
    src = edge_index[0].astype(jnp.int32)
    dst = edge_index[1].astype(jnp.int32)
    emb0 = jnp.concatenate([user_emb.astype(f32), item_emb.astype(f32)], axis=0)
    emb0 = jnp.pad(emb0, ((0, NP - N), (0, 0)))
    asp_list = [pad_aspect[:, k, :].astype(f32) for k in range(MAXLEN)]

    wq = in_proj_w[:D].astype(f32)
    wk = in_proj_w[D:2 * D].astype(f32)
    wv = in_proj_w[2 * D:].astype(f32)
    bq = in_proj_b[:D].astype(f32).reshape(1, D)
    bk = in_proj_b[D:2 * D].astype(f32).reshape(1, D)
    bv = in_proj_b[2 * D:].astype(f32).reshape(1, D)
    hmat_np = np.zeros((D, HEADS), np.float32)
    for h_ in range(HEADS):
        hmat_np[h_ * DH:(h_ + 1) * DH, h_] = 1.0
    hmat = jnp.asarray(hmat_np)
    emat = jnp.asarray(np.ascontiguousarray(hmat_np.T))
    weights = (wq.T, wk.T, wv.T, bq, bk, bv,
               out_proj_w.astype(f32).T, out_proj_b.astype(f32).reshape(1, D),
               hmat, emat)

    pad_n = _EP - E
    src_p = jnp.concatenate(
        [src, (jnp.arange(pad_n, dtype=jnp.int32) % jnp.int32(N))])
    dst_p = jnp.concatenate(
        [dst, jnp.full((pad_n,), -1, dtype=jnp.int32)])

    rsrc, rdst, rcnt = _route(src_p, dst_p)
    deg_out, deg_in = _bincounts(src, dst)
    deg_out2 = deg_out.reshape(NP, 1)
    deg_in2 = deg_in.reshape(NP, 1)

    h = _tc_prep(emb0, deg_out2)
    acc = emb0
    for layer in range(LAYERS):
        agg = _segsum(h, rsrc, rdst, rcnt)
        acc, h = _tc_layer(agg, deg_in2, deg_out2, asp_list, weights, acc,
                           last=(layer == LAYERS - 1))
    light = acc[:N]
    return (light[:N_USERS], light[N_USERS:])
